# baseline (speedup 1.0000x reference)
import jax
import jax.numpy as jnp
from jax.experimental import pallas as pl
from jax.experimental.pallas import tpu as pltpu

_GRID = 0.01


def _fused_kernel(x_ref, whi_ref, wlo_ref, bias_ref, w1_ref, b1_ref, w2_ref,
                  b2_ref, w3_ref, b3_ref, o_ref, acc_ref):
    b = pl.program_id(0)
    nb = pl.num_programs(0)
    xt = x_ref[0]
    ci = jnp.floor(xt / _GRID).astype(jnp.bfloat16)
    dn = (((0,), (0,)), ((), ()))
    h = jax.lax.dot_general(ci, whi_ref[:], dn,
                            preferred_element_type=jnp.float32)
    h = h + jax.lax.dot_general(ci, wlo_ref[:], dn,
                                preferred_element_type=jnp.float32)
    acc_ref[pl.ds(b, 1), :] = jnp.max(h, axis=0, keepdims=True)

    @pl.when(b == nb - 1)
    def _mlp():
        pooled = jnp.maximum(acc_ref[:, :] + bias_ref[:], 0.0)
        x1 = jnp.maximum(
            jnp.dot(pooled, w1_ref[:], preferred_element_type=jnp.float32)
            + b1_ref[:], 0.0)
        x2 = jnp.maximum(
            jnp.dot(x1, w2_ref[:], preferred_element_type=jnp.float32)
            + b2_ref[:], 0.0)
        o_ref[:] = (
            jnp.dot(x2, w3_ref[:], preferred_element_type=jnp.float32)
            + b3_ref[:])


def kernel(input, W_enc, b_enc, W1, b1, W2, b2, W3, b3):
    if input.shape[-1] != 3:
        input = jnp.transpose(input, (0, 2, 1))
    B, N = input.shape[0], input.shape[1]
    F = W_enc.shape[1]
    H1, H2, P = W1.shape[1], W2.shape[1], W3.shape[1]
    PP = 128

    xt = jnp.transpose(input, (0, 2, 1))
    wg = W_enc[1:4] * _GRID
    w_hi = wg.astype(jnp.bfloat16)
    w_lo = (wg - w_hi.astype(jnp.float32)).astype(jnp.bfloat16)
    bias0 = (b_enc + W_enc[0]).reshape(1, F)
    W3p = jnp.pad(W3, ((0, 0), (0, PP - P)))
    b3p = jnp.pad(b3, (0, PP - P)).reshape(1, PP)

    pose = pl.pallas_call(
        _fused_kernel,
        grid=(B,),
        in_specs=[
            pl.BlockSpec((1, 3, N), lambda b: (b, 0, 0)),
            pl.BlockSpec((3, F), lambda b: (0, 0)),
            pl.BlockSpec((3, F), lambda b: (0, 0)),
            pl.BlockSpec((1, F), lambda b: (0, 0)),
            pl.BlockSpec((F, H1), lambda b: (0, 0)),
            pl.BlockSpec((1, H1), lambda b: (0, 0)),
            pl.BlockSpec((H1, H2), lambda b: (0, 0)),
            pl.BlockSpec((1, H2), lambda b: (0, 0)),
            pl.BlockSpec((H2, PP), lambda b: (0, 0)),
            pl.BlockSpec((1, PP), lambda b: (0, 0)),
        ],
        out_specs=pl.BlockSpec((B, PP), lambda b: (0, 0)),
        out_shape=jax.ShapeDtypeStruct((B, PP), jnp.float32),
        scratch_shapes=[pltpu.VMEM((B, F), jnp.float32)],
    )(xt, w_hi, w_lo, bias0, W1, b1.reshape(1, H1), W2, b2.reshape(1, H2),
      W3p, b3p)

    return pose[:, :P]


# speedup vs baseline: 1.6659x; 1.6659x over previous
import jax
import jax.numpy as jnp
from jax.experimental import pallas as pl
from jax.experimental.pallas import tpu as pltpu

_GRID = 0.01


def _fused_kernel(x_ref, w_ref, bias_ref, w1_ref, b1_ref, w2_ref,
                  b2_ref, w3_ref, b3_ref, o_ref, acc_ref):
    b = pl.program_id(0)
    nb = pl.num_programs(0)
    xt = x_ref[0]
    ci = jnp.floor(xt / _GRID).astype(jnp.bfloat16)
    ci2 = jnp.concatenate([ci, ci], axis=0)
    h = jax.lax.dot_general(ci2, w_ref[:], (((0,), (0,)), ((), ())),
                            preferred_element_type=jnp.float32)
    acc_ref[pl.ds(b, 1), :] = jnp.max(h, axis=0, keepdims=True)

    @pl.when(b == nb - 1)
    def _mlp():
        pooled = jnp.maximum(acc_ref[:, :] + bias_ref[:], 0.0)
        x1 = jnp.maximum(
            jnp.dot(pooled, w1_ref[:], preferred_element_type=jnp.float32)
            + b1_ref[:], 0.0)
        x2 = jnp.maximum(
            jnp.dot(x1, w2_ref[:], preferred_element_type=jnp.float32)
            + b2_ref[:], 0.0)
        o_ref[:] = (
            jnp.dot(x2, w3_ref[:], preferred_element_type=jnp.float32)
            + b3_ref[:])


def kernel(input, W_enc, b_enc, W1, b1, W2, b2, W3, b3):
    if input.shape[-1] != 3:
        input = jnp.transpose(input, (0, 2, 1))
    B, N = input.shape[0], input.shape[1]
    F = W_enc.shape[1]
    H1, H2, P = W1.shape[1], W2.shape[1], W3.shape[1]
    PP = 128

    xt = jnp.transpose(input, (0, 2, 1))
    wg = W_enc[1:4] * _GRID
    w_hi = wg.astype(jnp.bfloat16)
    w_lo = (wg - w_hi.astype(jnp.float32)).astype(jnp.bfloat16)
    w_cat = jnp.concatenate([w_hi, w_lo], axis=0)
    bias0 = (b_enc + W_enc[0]).reshape(1, F)
    W3p = jnp.pad(W3, ((0, 0), (0, PP - P)))
    b3p = jnp.pad(b3, (0, PP - P)).reshape(1, PP)

    pose = pl.pallas_call(
        _fused_kernel,
        grid=(B,),
        in_specs=[
            pl.BlockSpec((1, 3, N), lambda b: (b, 0, 0)),
            pl.BlockSpec((6, F), lambda b: (0, 0)),
            pl.BlockSpec((1, F), lambda b: (0, 0)),
            pl.BlockSpec((F, H1), lambda b: (0, 0)),
            pl.BlockSpec((1, H1), lambda b: (0, 0)),
            pl.BlockSpec((H1, H2), lambda b: (0, 0)),
            pl.BlockSpec((1, H2), lambda b: (0, 0)),
            pl.BlockSpec((H2, PP), lambda b: (0, 0)),
            pl.BlockSpec((1, PP), lambda b: (0, 0)),
        ],
        out_specs=pl.BlockSpec((B, PP), lambda b: (0, 0)),
        out_shape=jax.ShapeDtypeStruct((B, PP), jnp.float32),
        scratch_shapes=[pltpu.VMEM((B, F), jnp.float32)],
    )(xt, w_cat, bias0, W1, b1.reshape(1, H1), W2, b2.reshape(1, H2),
      W3p, b3p)

    return pose[:, :P]
